# Initial kernel scaffold; baseline (speedup 1.0000x reference)
import jax
import jax.numpy as jnp
from jax.experimental import pallas as pl


def kernel(x, weight, bias, gamma, beta):
    raise NotImplementedError("write your pallas kernel here")


# speedup vs baseline: 1.0931x; 1.0931x over previous
import functools
import math

import jax
import jax.numpy as jnp
from jax import lax
from jax.experimental import pallas as pl
from jax.experimental.pallas import tpu as pltpu

_BN_EPS = 1e-5
_INV_SQRT2 = 0.7071067811865476


def _gelu_exact(z):
    return 0.5 * z * (1.0 + lax.erf(z * _INV_SQRT2))


def _pick_block(n):
    for b in (4, 2, 1):
        if n % b == 0:
            return b
    return 1


def _gram_kernel(x_ref, g_ref, s_ref, *, b_blk):
    @pl.when(pl.program_id(1) == 0)
    def _():
        g_ref[...] = jnp.zeros_like(g_ref)
        s_ref[...] = jnp.zeros_like(s_ref)

    g = jnp.zeros(g_ref.shape[1:], jnp.float32)
    s = jnp.zeros(s_ref.shape[1:], jnp.float32)
    for b in range(b_blk):
        xb = x_ref[b]
        xb16 = xb.astype(jnp.bfloat16)
        g = g + lax.dot_general(xb16, xb16, (((1,), (1,)), ((), ())),
                                preferred_element_type=jnp.float32)
        s = s + jnp.sum(xb, axis=1, keepdims=True)
    g_ref[0] += g
    s_ref[0] += s


def _apply_kernel(x_ref, w_ref, scale_ref, shift_ref, o_ref, *, b_blk):
    w = w_ref[...]
    scale = scale_ref[...]
    shift = shift_ref[...]
    for b in range(b_blk):
        y = jnp.dot(w, x_ref[b].astype(jnp.bfloat16),
                    preferred_element_type=jnp.float32)
        o_ref[b] = _gelu_exact(y * scale + shift).astype(o_ref.dtype)


def kernel(x, weight, bias, gamma, beta):
    del bias
    N, C_in, H, W = x.shape
    C_out = weight.shape[0]
    HW = H * W
    M = N * HW
    HW_pad = ((HW + 127) // 128) * 128
    out_dtype = x.dtype

    x3 = x.reshape(N, C_in, HW)
    if HW_pad != HW:
        x3 = jnp.pad(x3, ((0, 0), (0, 0), (0, HW_pad - HW)))

    n_cores = 2 if N % 2 == 0 else 1
    per_core = N // n_cores
    b1 = _pick_block(per_core)
    steps1 = per_core // b1

    g_parts, s_parts = pl.pallas_call(
        functools.partial(_gram_kernel, b_blk=b1),
        out_shape=(
            jax.ShapeDtypeStruct((n_cores, C_in, C_in), jnp.float32),
            jax.ShapeDtypeStruct((n_cores, C_in, 1), jnp.float32),
        ),
        grid=(n_cores, steps1),
        in_specs=[
            pl.BlockSpec((b1, C_in, HW_pad),
                         lambda c, i: (c * steps1 + i, 0, 0)),
        ],
        out_specs=(
            pl.BlockSpec((1, C_in, C_in), lambda c, i: (c, 0, 0)),
            pl.BlockSpec((1, C_in, 1), lambda c, i: (c, 0, 0)),
        ),
        compiler_params=pltpu.CompilerParams(
            dimension_semantics=("parallel", "arbitrary")),
        cost_estimate=pl.CostEstimate(
            flops=2 * M * C_in * C_in + 2 * M * C_in,
            transcendentals=0,
            bytes_accessed=4 * N * C_in * HW_pad),
    )(x3)

    g = jnp.sum(g_parts, axis=0)
    s = jnp.sum(s_parts, axis=0)[:, 0]
    wf = weight.astype(jnp.float32)
    sum_y = jnp.dot(wf, s, precision=lax.Precision.HIGHEST)
    wg = jnp.dot(wf, g, precision=lax.Precision.HIGHEST)
    sum_y2 = jnp.sum(wg * wf, axis=1)
    inv_m = 1.0 / float(M)
    mean = sum_y * inv_m
    var = jnp.maximum(sum_y2 * inv_m - mean * mean, 0.0)
    scale = gamma.astype(jnp.float32) * lax.rsqrt(var + _BN_EPS)
    shift = beta.astype(jnp.float32) - mean * scale
    scale2 = scale.reshape(C_out, 1)
    shift2 = shift.reshape(C_out, 1)
    w16 = weight.astype(jnp.bfloat16)

    b2 = _pick_block(per_core)
    steps2 = per_core // b2
    out3 = pl.pallas_call(
        functools.partial(_apply_kernel, b_blk=b2),
        out_shape=jax.ShapeDtypeStruct((N, C_out, HW_pad), out_dtype),
        grid=(n_cores, steps2),
        in_specs=[
            pl.BlockSpec((b2, C_in, HW_pad),
                         lambda c, i: (c * steps2 + i, 0, 0)),
            pl.BlockSpec((C_out, C_in), lambda c, i: (0, 0)),
            pl.BlockSpec((C_out, 1), lambda c, i: (0, 0)),
            pl.BlockSpec((C_out, 1), lambda c, i: (0, 0)),
        ],
        out_specs=pl.BlockSpec((b2, C_out, HW_pad),
                               lambda c, i: (c * steps2 + i, 0, 0)),
        compiler_params=pltpu.CompilerParams(
            dimension_semantics=("parallel", "arbitrary")),
        cost_estimate=pl.CostEstimate(
            flops=2 * M * C_in * C_out + 8 * M * C_out,
            transcendentals=M * C_out,
            bytes_accessed=4 * N * HW_pad * (C_in + C_out)),
    )(x3, w16, scale2, shift2)

    if HW_pad != HW:
        out3 = out3[:, :, :HW]
    return out3.reshape(N, C_out, H, W)


# speedup vs baseline: 2.6228x; 2.3993x over previous
import functools
import math

import jax
import jax.numpy as jnp
from jax import lax
from jax.experimental import pallas as pl
from jax.experimental.pallas import tpu as pltpu

_BN_EPS = 1e-5
_INV_SQRT2 = 0.7071067811865476


def _gelu_exact(z):
    return 0.5 * z * (1.0 + lax.erf(z * _INV_SQRT2))


def _pick_block(n):
    for b in (4, 2, 1):
        if n % b == 0:
            return b
    return 1


def _gram_kernel(x_ref, g_ref, s_ref, *, b_blk):
    @pl.when(pl.program_id(1) == 0)
    def _():
        g_ref[...] = jnp.zeros_like(g_ref)
        s_ref[...] = jnp.zeros_like(s_ref)

    g = jnp.zeros(g_ref.shape[1:], jnp.float32)
    s = jnp.zeros(s_ref.shape[1:], jnp.float32)
    for b in range(b_blk):
        xb = x_ref[b]
        xb16 = xb.astype(jnp.bfloat16)
        g = g + lax.dot_general(xb16, xb16, (((0,), (0,)), ((), ())),
                                preferred_element_type=jnp.float32)
        s = s + jnp.sum(xb, axis=0, keepdims=True)
    g_ref[0] += g
    s_ref[0] += s


def _apply_kernel(x_ref, w_ref, scale_ref, shift_ref, o_ref, *, b_blk):
    w = w_ref[...]
    scale = scale_ref[...]
    shift = shift_ref[...]
    for b in range(b_blk):
        y = jnp.dot(x_ref[b].astype(jnp.bfloat16), w,
                    preferred_element_type=jnp.float32)
        o_ref[b] = _gelu_exact(y * scale + shift).astype(o_ref.dtype)


def kernel(x, weight, bias, gamma, beta):
    del bias
    N, C_in, H, W = x.shape
    C_out = weight.shape[0]
    HW = H * W
    M = N * HW
    HW_pad = ((HW + 7) // 8) * 8
    out_dtype = x.dtype

    xt = jnp.transpose(x.reshape(N, C_in, HW), (0, 2, 1))
    if HW_pad != HW:
        xt = jnp.pad(xt, ((0, 0), (0, HW_pad - HW), (0, 0)))

    n_cores = 2 if N % 2 == 0 else 1
    per_core = N // n_cores
    b1 = _pick_block(per_core)
    steps1 = per_core // b1

    g_parts, s_parts = pl.pallas_call(
        functools.partial(_gram_kernel, b_blk=b1),
        out_shape=(
            jax.ShapeDtypeStruct((n_cores, C_in, C_in), jnp.float32),
            jax.ShapeDtypeStruct((n_cores, 1, C_in), jnp.float32),
        ),
        grid=(n_cores, steps1),
        in_specs=[
            pl.BlockSpec((b1, HW_pad, C_in),
                         lambda c, i: (c * steps1 + i, 0, 0)),
        ],
        out_specs=(
            pl.BlockSpec((1, C_in, C_in), lambda c, i: (c, 0, 0)),
            pl.BlockSpec((1, 1, C_in), lambda c, i: (c, 0, 0)),
        ),
        compiler_params=pltpu.CompilerParams(
            dimension_semantics=("parallel", "arbitrary")),
        cost_estimate=pl.CostEstimate(
            flops=2 * M * C_in * C_in + 2 * M * C_in,
            transcendentals=0,
            bytes_accessed=4 * N * C_in * HW_pad),
    )(xt)

    g = jnp.sum(g_parts, axis=0)
    s = jnp.sum(s_parts, axis=0)[0]
    wf = weight.astype(jnp.float32)
    sum_y = jnp.dot(wf, s, precision=lax.Precision.HIGHEST)
    wg = jnp.dot(wf, g, precision=lax.Precision.HIGHEST)
    sum_y2 = jnp.sum(wg * wf, axis=1)
    inv_m = 1.0 / float(M)
    mean = sum_y * inv_m
    var = jnp.maximum(sum_y2 * inv_m - mean * mean, 0.0)
    scale = gamma.astype(jnp.float32) * lax.rsqrt(var + _BN_EPS)
    shift = beta.astype(jnp.float32) - mean * scale
    scale2 = scale.reshape(1, C_out)
    shift2 = shift.reshape(1, C_out)
    w16t = weight.astype(jnp.bfloat16).T

    b2 = _pick_block(per_core)
    steps2 = per_core // b2
    ot = pl.pallas_call(
        functools.partial(_apply_kernel, b_blk=b2),
        out_shape=jax.ShapeDtypeStruct((N, HW_pad, C_out), out_dtype),
        grid=(n_cores, steps2),
        in_specs=[
            pl.BlockSpec((b2, HW_pad, C_in),
                         lambda c, i: (c * steps2 + i, 0, 0)),
            pl.BlockSpec((C_in, C_out), lambda c, i: (0, 0)),
            pl.BlockSpec((1, C_out), lambda c, i: (0, 0)),
            pl.BlockSpec((1, C_out), lambda c, i: (0, 0)),
        ],
        out_specs=pl.BlockSpec((b2, HW_pad, C_out),
                               lambda c, i: (c * steps2 + i, 0, 0)),
        compiler_params=pltpu.CompilerParams(
            dimension_semantics=("parallel", "arbitrary")),
        cost_estimate=pl.CostEstimate(
            flops=2 * M * C_in * C_out + 8 * M * C_out,
            transcendentals=M * C_out,
            bytes_accessed=4 * N * HW_pad * (C_in + C_out)),
    )(xt, w16t, scale2, shift2)

    if HW_pad != HW:
        ot = ot[:, :HW, :]
    return jnp.transpose(ot, (0, 2, 1)).reshape(N, C_out, H, W)


# speedup vs baseline: 2.9922x; 1.1408x over previous
import functools
import math

import jax
import jax.numpy as jnp
from jax import lax
from jax.experimental import pallas as pl
from jax.experimental.pallas import tpu as pltpu

_BN_EPS = 1e-5
_INV_SQRT2 = 0.7071067811865476


def _gelu_exact(z):
    return 0.5 * z * (1.0 + lax.erf(z * _INV_SQRT2))


def _pick_block(n, pref=(8, 4, 2, 1)):
    for b in pref:
        if n % b == 0:
            return b
    return 1


def _gram_kernel(x_ref, g_ref, s_ref, *, b_blk):
    @pl.when(pl.program_id(1) == 0)
    def _():
        g_ref[...] = jnp.zeros_like(g_ref)
        s_ref[...] = jnp.zeros_like(s_ref)

    g = jnp.zeros(g_ref.shape[1:], jnp.float32)
    s = jnp.zeros(s_ref.shape[1:], jnp.float32)
    for b in range(b_blk):
        xb = x_ref[b]
        xb16 = xb.astype(jnp.bfloat16)
        g = g + lax.dot_general(xb16, xb16, (((0,), (0,)), ((), ())),
                                preferred_element_type=jnp.float32)
        s = s + jnp.sum(xb, axis=0, keepdims=True)
    g_ref[0] += g
    s_ref[0] += s


def _apply_kernel(x_ref, w_ref, scale_ref, shift_ref, o_ref, *, b_blk):
    w = w_ref[...]
    scale = scale_ref[...]
    shift = shift_ref[...]
    for b in range(b_blk):
        y = jnp.dot(x_ref[b].astype(jnp.bfloat16), w,
                    preferred_element_type=jnp.float32)
        o_ref[b] = _gelu_exact(y * scale + shift).astype(o_ref.dtype)


def kernel(x, weight, bias, gamma, beta):
    del bias
    N, C_in, H, W = x.shape
    C_out = weight.shape[0]
    HW = H * W
    M = N * HW
    HW_pad = ((HW + 7) // 8) * 8
    out_dtype = x.dtype

    xt = jnp.transpose(x.reshape(N, C_in, HW), (0, 2, 1))
    if HW_pad != HW:
        xt = jnp.pad(xt, ((0, 0), (0, HW_pad - HW), (0, 0)))

    n_cores = 2 if N % 2 == 0 else 1
    per_core = N // n_cores
    b1 = _pick_block(per_core)
    steps1 = per_core // b1

    g_parts, s_parts = pl.pallas_call(
        functools.partial(_gram_kernel, b_blk=b1),
        out_shape=(
            jax.ShapeDtypeStruct((n_cores, C_in, C_in), jnp.float32),
            jax.ShapeDtypeStruct((n_cores, 1, C_in), jnp.float32),
        ),
        grid=(n_cores, steps1),
        in_specs=[
            pl.BlockSpec((b1, HW_pad, C_in),
                         lambda c, i: (c * steps1 + i, 0, 0)),
        ],
        out_specs=(
            pl.BlockSpec((1, C_in, C_in), lambda c, i: (c, 0, 0)),
            pl.BlockSpec((1, 1, C_in), lambda c, i: (c, 0, 0)),
        ),
        compiler_params=pltpu.CompilerParams(
            dimension_semantics=("parallel", "arbitrary")),
        cost_estimate=pl.CostEstimate(
            flops=2 * M * C_in * C_in + 2 * M * C_in,
            transcendentals=0,
            bytes_accessed=4 * N * C_in * HW_pad),
    )(xt)

    g = jnp.sum(g_parts, axis=0)
    s = jnp.sum(s_parts, axis=0)[0]
    wf = weight.astype(jnp.float32)
    sum_y = jnp.dot(wf, s, precision=lax.Precision.HIGHEST)
    wg = jnp.dot(wf, g, precision=lax.Precision.HIGHEST)
    sum_y2 = jnp.sum(wg * wf, axis=1)
    inv_m = 1.0 / float(M)
    mean = sum_y * inv_m
    var = jnp.maximum(sum_y2 * inv_m - mean * mean, 0.0)
    scale = gamma.astype(jnp.float32) * lax.rsqrt(var + _BN_EPS)
    shift = beta.astype(jnp.float32) - mean * scale
    scale2 = scale.reshape(1, C_out)
    shift2 = shift.reshape(1, C_out)
    w16t = weight.astype(jnp.bfloat16).T

    b2 = _pick_block(per_core)
    steps2 = per_core // b2
    ot = pl.pallas_call(
        functools.partial(_apply_kernel, b_blk=b2),
        out_shape=jax.ShapeDtypeStruct((N, HW_pad, C_out), out_dtype),
        grid=(n_cores, steps2),
        in_specs=[
            pl.BlockSpec((b2, HW_pad, C_in),
                         lambda c, i: (c * steps2 + i, 0, 0)),
            pl.BlockSpec((C_in, C_out), lambda c, i: (0, 0)),
            pl.BlockSpec((1, C_out), lambda c, i: (0, 0)),
            pl.BlockSpec((1, C_out), lambda c, i: (0, 0)),
        ],
        out_specs=pl.BlockSpec((b2, HW_pad, C_out),
                               lambda c, i: (c * steps2 + i, 0, 0)),
        compiler_params=pltpu.CompilerParams(
            dimension_semantics=("parallel", "arbitrary")),
        cost_estimate=pl.CostEstimate(
            flops=2 * M * C_in * C_out + 8 * M * C_out,
            transcendentals=M * C_out,
            bytes_accessed=4 * N * HW_pad * (C_in + C_out)),
    )(xt, w16t, scale2, shift2)

    if HW_pad != HW:
        ot = ot[:, :HW, :]
    return jnp.transpose(ot, (0, 2, 1)).reshape(N, C_out, H, W)


# speedup vs baseline: 3.3648x; 1.1245x over previous
import functools
import math

import jax
import jax.numpy as jnp
from jax import lax
from jax.experimental import pallas as pl
from jax.experimental.pallas import tpu as pltpu

_BN_EPS = 1e-5
_INV_SQRT2 = 0.7071067811865476


def _gelu_exact(z):
    return 0.5 * z * (1.0 + lax.erf(z * _INV_SQRT2))


def _pick_block(n, pref):
    for b in pref:
        if n % b == 0:
            return b
    return 1


def _gram_kernel(x_ref, g_ref, s_ref, *, b_blk):
    @pl.when(pl.program_id(1) == 0)
    def _():
        g_ref[...] = jnp.zeros_like(g_ref)
        s_ref[...] = jnp.zeros_like(s_ref)

    g = jnp.zeros(g_ref.shape[1:], jnp.float32)
    s = jnp.zeros(s_ref.shape[1:], jnp.float32)
    for b in range(b_blk):
        xb = x_ref[b]
        xb16 = xb.astype(jnp.bfloat16)
        g = g + lax.dot_general(xb16, xb16, (((0,), (0,)), ((), ())),
                                preferred_element_type=jnp.float32)
        s = s + jnp.sum(xb, axis=0, keepdims=True)
    g_ref[0] += g
    s_ref[0] += s


def _apply_kernel(x_ref, wt_ref, g_ref, s_ref, gamma_ref, beta_ref, o_ref,
                  w16_ref, scale_ref, shift_ref, *, b_blk, inv_m):
    @pl.when(pl.program_id(1) == 0)
    def _():
        wt = wt_ref[...]
        w16_ref[...] = wt.astype(jnp.bfloat16)
        g = jnp.sum(g_ref[...], axis=0)
        s = jnp.sum(s_ref[...], axis=0)
        gw = jnp.dot(g, wt, preferred_element_type=jnp.float32)
        sum_y2 = jnp.sum(gw * wt, axis=0, keepdims=True)
        sum_y = jnp.dot(s, wt, preferred_element_type=jnp.float32)
        mean = sum_y * inv_m
        var = jnp.maximum(sum_y2 * inv_m - mean * mean, 0.0)
        scale = gamma_ref[...] * lax.rsqrt(var + _BN_EPS)
        scale_ref[...] = scale
        shift_ref[...] = beta_ref[...] - mean * scale

    w16 = w16_ref[...]
    scale = scale_ref[...]
    shift = shift_ref[...]
    for b in range(b_blk):
        y = jnp.dot(x_ref[b].astype(jnp.bfloat16), w16,
                    preferred_element_type=jnp.float32)
        o_ref[b] = _gelu_exact(y * scale + shift).astype(o_ref.dtype)


def kernel(x, weight, bias, gamma, beta):
    del bias
    N, C_in, H, W = x.shape
    C_out = weight.shape[0]
    HW = H * W
    M = N * HW
    HW_pad = ((HW + 7) // 8) * 8
    out_dtype = x.dtype
    inv_m = 1.0 / float(M)

    xt = jnp.transpose(x.reshape(N, C_in, HW), (0, 2, 1))
    if HW_pad != HW:
        xt = jnp.pad(xt, ((0, 0), (0, HW_pad - HW), (0, 0)))

    n_cores = 2 if N % 2 == 0 else 1
    per_core = N // n_cores
    b1 = _pick_block(per_core, (16, 8, 4, 2, 1))
    steps1 = per_core // b1

    g_parts, s_parts = pl.pallas_call(
        functools.partial(_gram_kernel, b_blk=b1),
        out_shape=(
            jax.ShapeDtypeStruct((n_cores, C_in, C_in), jnp.float32),
            jax.ShapeDtypeStruct((n_cores, 1, C_in), jnp.float32),
        ),
        grid=(n_cores, steps1),
        in_specs=[
            pl.BlockSpec((b1, HW_pad, C_in),
                         lambda c, i: (c * steps1 + i, 0, 0)),
        ],
        out_specs=(
            pl.BlockSpec((1, C_in, C_in), lambda c, i: (c, 0, 0)),
            pl.BlockSpec((1, 1, C_in), lambda c, i: (c, 0, 0)),
        ),
        compiler_params=pltpu.CompilerParams(
            dimension_semantics=("parallel", "arbitrary")),
        cost_estimate=pl.CostEstimate(
            flops=2 * M * C_in * C_in + 2 * M * C_in,
            transcendentals=0,
            bytes_accessed=4 * N * C_in * HW_pad),
    )(xt)

    wt = weight.astype(jnp.float32).T
    gamma2 = gamma.astype(jnp.float32).reshape(1, C_out)
    beta2 = beta.astype(jnp.float32).reshape(1, C_out)

    b2 = _pick_block(per_core, (8, 4, 2, 1))
    steps2 = per_core // b2
    ot = pl.pallas_call(
        functools.partial(_apply_kernel, b_blk=b2, inv_m=inv_m),
        out_shape=jax.ShapeDtypeStruct((N, HW_pad, C_out), out_dtype),
        grid=(n_cores, steps2),
        in_specs=[
            pl.BlockSpec((b2, HW_pad, C_in),
                         lambda c, i: (c * steps2 + i, 0, 0)),
            pl.BlockSpec((C_in, C_out), lambda c, i: (0, 0)),
            pl.BlockSpec((n_cores, C_in, C_in), lambda c, i: (0, 0, 0)),
            pl.BlockSpec((n_cores, 1, C_in), lambda c, i: (0, 0, 0)),
            pl.BlockSpec((1, C_out), lambda c, i: (0, 0)),
            pl.BlockSpec((1, C_out), lambda c, i: (0, 0)),
        ],
        out_specs=pl.BlockSpec((b2, HW_pad, C_out),
                               lambda c, i: (c * steps2 + i, 0, 0)),
        scratch_shapes=[
            pltpu.VMEM((C_in, C_out), jnp.bfloat16),
            pltpu.VMEM((1, C_out), jnp.float32),
            pltpu.VMEM((1, C_out), jnp.float32),
        ],
        compiler_params=pltpu.CompilerParams(
            dimension_semantics=("parallel", "arbitrary")),
        cost_estimate=pl.CostEstimate(
            flops=2 * M * C_in * C_out + 8 * M * C_out,
            transcendentals=M * C_out,
            bytes_accessed=4 * N * HW_pad * (C_in + C_out)),
    )(xt, wt, g_parts, s_parts, gamma2, beta2)

    if HW_pad != HW:
        ot = ot[:, :HW, :]
    return jnp.transpose(ot, (0, 2, 1)).reshape(N, C_out, H, W)


# speedup vs baseline: 3.4905x; 1.0374x over previous
import functools
import math

import jax
import jax.numpy as jnp
from jax import lax
from jax.experimental import pallas as pl
from jax.experimental.pallas import tpu as pltpu

_BN_EPS = 1e-5
_INV_SQRT2 = 0.7071067811865476


def _gelu_exact(z):
    return 0.5 * z * (1.0 + lax.erf(z * _INV_SQRT2))


def _pick_block(n, pref):
    for b in pref:
        if n % b == 0:
            return b
    return 1


def _gram_kernel(x_ref, g_ref, s_ref, *, b_blk):
    @pl.when(pl.program_id(1) == 0)
    def _():
        g_ref[...] = jnp.zeros_like(g_ref)
        s_ref[...] = jnp.zeros_like(s_ref)

    g = jnp.zeros(g_ref.shape[1:], jnp.float32)
    s = jnp.zeros(s_ref.shape[1:], jnp.float32)
    for b in range(b_blk):
        xb = x_ref[b]
        xb16 = xb.astype(jnp.bfloat16)
        g = g + lax.dot_general(xb16, xb16, (((0,), (0,)), ((), ())),
                                preferred_element_type=jnp.float32)
        s = s + jnp.sum(xb, axis=0, keepdims=True)
    g_ref[0] += g
    s_ref[0] += s


def _apply_kernel(x_ref, w_ref, g_ref, s_ref, gamma_ref, beta_ref, o_ref,
                  w16_ref, scale_ref, shift_ref, *, b_blk, inv_m):
    @pl.when(pl.program_id(1) == 0)
    def _():
        w = w_ref[...]
        w16_ref[...] = w.astype(jnp.bfloat16)
        g = jnp.sum(g_ref[...], axis=0)
        s = jnp.sum(s_ref[...], axis=0)
        gw = lax.dot_general(g, w, (((1,), (1,)), ((), ())),
                             preferred_element_type=jnp.float32)
        sum_y2 = jnp.sum(gw * w.T, axis=0, keepdims=True)
        sum_y = lax.dot_general(s, w, (((1,), (1,)), ((), ())),
                                preferred_element_type=jnp.float32)
        mean = sum_y * inv_m
        var = jnp.maximum(sum_y2 * inv_m - mean * mean, 0.0)
        scale = gamma_ref[...] * lax.rsqrt(var + _BN_EPS)
        scale_ref[...] = scale
        shift_ref[...] = beta_ref[...] - mean * scale

    w16 = w16_ref[...]
    scale = scale_ref[...]
    shift = shift_ref[...]
    for b in range(b_blk):
        y = lax.dot_general(x_ref[b].astype(jnp.bfloat16), w16,
                            (((1,), (1,)), ((), ())),
                            preferred_element_type=jnp.float32)
        o_ref[b] = _gelu_exact(y * scale + shift).astype(o_ref.dtype)


def kernel(x, weight, bias, gamma, beta):
    del bias
    N, C_in, H, W = x.shape
    C_out = weight.shape[0]
    HW = H * W
    M = N * HW
    HW_pad = ((HW + 7) // 8) * 8
    out_dtype = x.dtype
    inv_m = 1.0 / float(M)

    xt = jnp.transpose(x.reshape(N, C_in, HW), (0, 2, 1))
    if HW_pad != HW:
        xt = jnp.pad(xt, ((0, 0), (0, HW_pad - HW), (0, 0)))

    n_cores = 2 if N % 2 == 0 else 1
    per_core = N // n_cores
    b1 = _pick_block(per_core, (16, 8, 4, 2, 1))
    steps1 = per_core // b1

    g_parts, s_parts = pl.pallas_call(
        functools.partial(_gram_kernel, b_blk=b1),
        out_shape=(
            jax.ShapeDtypeStruct((n_cores, C_in, C_in), jnp.float32),
            jax.ShapeDtypeStruct((n_cores, 1, C_in), jnp.float32),
        ),
        grid=(n_cores, steps1),
        in_specs=[
            pl.BlockSpec((b1, HW_pad, C_in),
                         lambda c, i: (c * steps1 + i, 0, 0)),
        ],
        out_specs=(
            pl.BlockSpec((1, C_in, C_in), lambda c, i: (c, 0, 0)),
            pl.BlockSpec((1, 1, C_in), lambda c, i: (c, 0, 0)),
        ),
        compiler_params=pltpu.CompilerParams(
            dimension_semantics=("parallel", "arbitrary")),
        cost_estimate=pl.CostEstimate(
            flops=2 * M * C_in * C_in + 2 * M * C_in,
            transcendentals=0,
            bytes_accessed=4 * N * C_in * HW_pad),
    )(xt)

    wf = weight.astype(jnp.float32)
    gamma2 = gamma.astype(jnp.float32).reshape(1, C_out)
    beta2 = beta.astype(jnp.float32).reshape(1, C_out)

    b2 = _pick_block(per_core, (8, 4, 2, 1))
    steps2 = per_core // b2
    ot = pl.pallas_call(
        functools.partial(_apply_kernel, b_blk=b2, inv_m=inv_m),
        out_shape=jax.ShapeDtypeStruct((N, HW_pad, C_out), out_dtype),
        grid=(n_cores, steps2),
        in_specs=[
            pl.BlockSpec((b2, HW_pad, C_in),
                         lambda c, i: (c * steps2 + i, 0, 0)),
            pl.BlockSpec((C_out, C_in), lambda c, i: (0, 0)),
            pl.BlockSpec((n_cores, C_in, C_in), lambda c, i: (0, 0, 0)),
            pl.BlockSpec((n_cores, 1, C_in), lambda c, i: (0, 0, 0)),
            pl.BlockSpec((1, C_out), lambda c, i: (0, 0)),
            pl.BlockSpec((1, C_out), lambda c, i: (0, 0)),
        ],
        out_specs=pl.BlockSpec((b2, HW_pad, C_out),
                               lambda c, i: (c * steps2 + i, 0, 0)),
        scratch_shapes=[
            pltpu.VMEM((C_out, C_in), jnp.bfloat16),
            pltpu.VMEM((1, C_out), jnp.float32),
            pltpu.VMEM((1, C_out), jnp.float32),
        ],
        compiler_params=pltpu.CompilerParams(
            dimension_semantics=("parallel", "arbitrary")),
        cost_estimate=pl.CostEstimate(
            flops=2 * M * C_in * C_out + 8 * M * C_out,
            transcendentals=M * C_out,
            bytes_accessed=4 * N * HW_pad * (C_in + C_out)),
    )(xt, wf, g_parts, s_parts, gamma2, beta2)

    if HW_pad != HW:
        ot = ot[:, :HW, :]
    return jnp.transpose(ot, (0, 2, 1)).reshape(N, C_out, H, W)


# speedup vs baseline: 3.5862x; 1.0274x over previous
import functools
import math

import jax
import jax.numpy as jnp
from jax import lax
from jax.experimental import pallas as pl
from jax.experimental.pallas import tpu as pltpu

_BN_EPS = 1e-5
_INV_SQRT2 = 0.7071067811865476


def _gelu_exact(z):
    return 0.5 * z * (1.0 + lax.erf(z * _INV_SQRT2))


def _pick_block(n, pref):
    for b in pref:
        if n % b == 0:
            return b
    return 1


def _gram_kernel(x_ref, g_ref, s_ref, *, b_blk):
    @pl.when(pl.program_id(1) == 0)
    def _():
        g_ref[...] = jnp.zeros_like(g_ref)
        s_ref[...] = jnp.zeros_like(s_ref)

    g = jnp.zeros(g_ref.shape[1:], jnp.float32)
    s = jnp.zeros(s_ref.shape[1:], jnp.float32)
    for b in range(b_blk):
        xb = x_ref[b]
        xb16 = xb.astype(jnp.bfloat16)
        g = g + lax.dot_general(xb16, xb16, (((0,), (0,)), ((), ())),
                                preferred_element_type=jnp.float32)
        s = s + jnp.sum(xb, axis=0, keepdims=True)
    g_ref[0] += g
    s_ref[0] += s


def _apply_kernel(x_ref, w_ref, g_ref, s_ref, gamma_ref, beta_ref, o_ref,
                  w16_ref, scale_ref, shift_ref, *, b_blk, inv_m):
    @pl.when(pl.program_id(1) == 0)
    def _():
        w = w_ref[...]
        w16_ref[...] = w.astype(jnp.bfloat16)
        g = jnp.sum(g_ref[...], axis=0)
        s = jnp.sum(s_ref[...], axis=0)
        gw = lax.dot_general(g, w, (((1,), (1,)), ((), ())),
                             preferred_element_type=jnp.float32)
        sum_y2 = jnp.sum(gw * w.T, axis=0, keepdims=True)
        sum_y = lax.dot_general(s, w, (((1,), (1,)), ((), ())),
                                preferred_element_type=jnp.float32)
        mean = sum_y * inv_m
        var = jnp.maximum(sum_y2 * inv_m - mean * mean, 0.0)
        scale = gamma_ref[...] * lax.rsqrt(var + _BN_EPS)
        scale_ref[...] = scale
        shift_ref[...] = beta_ref[...] - mean * scale

    w16 = w16_ref[...]
    scale = scale_ref[...]
    shift = shift_ref[...]
    for b in range(b_blk):
        y = lax.dot_general(x_ref[b].astype(jnp.bfloat16), w16,
                            (((1,), (1,)), ((), ())),
                            preferred_element_type=jnp.float32)
        o_ref[b] = _gelu_exact(y * scale + shift).astype(o_ref.dtype)


def kernel(x, weight, bias, gamma, beta):
    del bias
    N, C_in, H, W = x.shape
    C_out = weight.shape[0]
    HW = H * W
    M = N * HW
    HW_pad = ((HW + 7) // 8) * 8
    out_dtype = x.dtype
    inv_m = 1.0 / float(M)

    xt = jnp.transpose(x.reshape(N, C_in, HW), (0, 2, 1))
    if HW_pad != HW:
        xt = jnp.pad(xt, ((0, 0), (0, HW_pad - HW), (0, 0)))

    n_cores = 2 if N % 2 == 0 else 1
    per_core = N // n_cores
    b1 = _pick_block(per_core, (16, 8, 4, 2, 1))
    steps1 = per_core // b1

    g_parts, s_parts = pl.pallas_call(
        functools.partial(_gram_kernel, b_blk=b1),
        out_shape=(
            jax.ShapeDtypeStruct((n_cores, C_in, C_in), jnp.float32),
            jax.ShapeDtypeStruct((n_cores, 1, C_in), jnp.float32),
        ),
        grid=(n_cores, steps1),
        in_specs=[
            pl.BlockSpec((b1, HW_pad, C_in),
                         lambda c, i: (c * steps1 + i, 0, 0)),
        ],
        out_specs=(
            pl.BlockSpec((1, C_in, C_in), lambda c, i: (c, 0, 0)),
            pl.BlockSpec((1, 1, C_in), lambda c, i: (c, 0, 0)),
        ),
        compiler_params=pltpu.CompilerParams(
            dimension_semantics=("parallel", "arbitrary")),
        cost_estimate=pl.CostEstimate(
            flops=2 * M * C_in * C_in + 2 * M * C_in,
            transcendentals=0,
            bytes_accessed=4 * N * C_in * HW_pad),
    )(xt)

    wf = weight.astype(jnp.float32)
    gamma2 = gamma.astype(jnp.float32).reshape(1, C_out)
    beta2 = beta.astype(jnp.float32).reshape(1, C_out)

    b2 = _pick_block(per_core, (16, 8, 4, 2, 1))
    steps2 = per_core // b2
    ot = pl.pallas_call(
        functools.partial(_apply_kernel, b_blk=b2, inv_m=inv_m),
        out_shape=jax.ShapeDtypeStruct((N, HW_pad, C_out), out_dtype),
        grid=(n_cores, steps2),
        in_specs=[
            pl.BlockSpec((b2, HW_pad, C_in),
                         lambda c, i: (c * steps2 + i, 0, 0)),
            pl.BlockSpec((C_out, C_in), lambda c, i: (0, 0)),
            pl.BlockSpec((n_cores, C_in, C_in), lambda c, i: (0, 0, 0)),
            pl.BlockSpec((n_cores, 1, C_in), lambda c, i: (0, 0, 0)),
            pl.BlockSpec((1, C_out), lambda c, i: (0, 0)),
            pl.BlockSpec((1, C_out), lambda c, i: (0, 0)),
        ],
        out_specs=pl.BlockSpec((b2, HW_pad, C_out),
                               lambda c, i: (c * steps2 + i, 0, 0)),
        scratch_shapes=[
            pltpu.VMEM((C_out, C_in), jnp.bfloat16),
            pltpu.VMEM((1, C_out), jnp.float32),
            pltpu.VMEM((1, C_out), jnp.float32),
        ],
        compiler_params=pltpu.CompilerParams(
            dimension_semantics=("parallel", "arbitrary"),
            vmem_limit_bytes=100 * 1024 * 1024),
        cost_estimate=pl.CostEstimate(
            flops=2 * M * C_in * C_out + 8 * M * C_out,
            transcendentals=M * C_out,
            bytes_accessed=4 * N * HW_pad * (C_in + C_out)),
    )(xt, wf, g_parts, s_parts, gamma2, beta2)

    if HW_pad != HW:
        ot = ot[:, :HW, :]
    return jnp.transpose(ot, (0, 2, 1)).reshape(N, C_out, H, W)
